# baseline (speedup 1.0000x reference)
import functools

import jax
import jax.numpy as jnp
from jax import lax
from jax.experimental import pallas as pl
from jax.experimental.pallas import tpu as pltpu
from jax.experimental.pallas import tpu_sc as plsc

NUM_CLASSES = 19
K = 10
KP = 16
D = 512
H, W = 270, 480
HW = H * W
FULL_H, FULL_W = 1080, 1920
FULL = FULL_H * FULL_W
N_SAMPLE = 135 * 240
NP = 32768
NEG = -1e30

NC, NS = 2, 16
NW = NC * NS
BPW = NP // NW
QTR = HW // 4
ZONE = FULL // NW


def argmax_body(o_ref, cls_ref):
    x = o_ref[...]
    m = jnp.max(x, axis=0, keepdims=True)
    cid = lax.broadcasted_iota(jnp.int32, x.shape, 0)
    first = jnp.min(jnp.where(x == m, cid, NUM_CLASSES), axis=0, keepdims=True)
    cls_ref[...] = first


def transpose_body(f_ref, o_ref):
    o_ref[...] = f_ref[...].T


def params_body(m_ref, lv_ref, lw_ref, M_ref, b_ref):
    mm = m_ref[...].reshape(NUM_CLASSES * K, D)
    lv = lv_ref[...].reshape(NUM_CLASSES * K, D)
    inv = jnp.exp(-lv)
    M_ref[...] = jnp.concatenate([-0.5 * inv, mm * inv], axis=-1)
    lw = lw_ref[...]
    mx = jnp.max(lw, axis=-1, keepdims=True)
    lse = mx + jnp.log(jnp.sum(jnp.exp(lw - mx), axis=-1, keepdims=True))
    q3 = jnp.sum(mm * mm * inv, axis=-1).reshape(NUM_CLASSES, K)
    lsv = jnp.sum(lv, axis=-1).reshape(NUM_CLASSES, K)
    b_ref[...] = (lw - lse) - 0.5 * (D * jnp.log(2.0 * jnp.pi) + lsv + q3)


def gmm_body(x_ref, cls_ref, Ms_ref, Mt_ref, aux_ref, pf_ref, pt_ref):
    xb = x_ref[...]
    xx = jnp.concatenate([xb * xb, xb], axis=1)
    dn = (((1,), (1,)), ((), ()))
    comp_s = lax.dot_general(xx, Ms_ref[...], dn,
                             preferred_element_type=jnp.float32)
    comp_t = lax.dot_general(xx, Mt_ref[...], dn,
                             preferred_element_type=jnp.float32)
    comp_s = comp_s + aux_ref[0:1, :]
    comp_t = comp_t + aux_ref[1:2, :]
    clsb = cls_ref[...]
    bn = xb.shape[0]
    sel_s = jnp.zeros((bn, KP), jnp.float32)
    sel_t = jnp.zeros((bn, KP), jnp.float32)
    cen = jnp.zeros((bn, KP), jnp.float32)
    for c in range(NUM_CLASSES):
        m = clsb == c
        sl = slice(c * KP, (c + 1) * KP)
        sel_s = jnp.where(m, comp_s[:, sl], sel_s)
        sel_t = jnp.where(m, comp_t[:, sl], sel_t)
        cen = jnp.where(m, aux_ref[2:3, sl], cen)
    mxs = jnp.max(sel_s, axis=1, keepdims=True)
    pf_ref[...] = mxs + jnp.log(
        jnp.sum(jnp.exp(sel_s - mxs), axis=1, keepdims=True))
    mxt = jnp.max(sel_t, axis=1, keepdims=True)
    kio = lax.broadcasted_iota(jnp.int32, (bn, KP), 1)
    am = jnp.min(jnp.where(sel_t == mxt, kio, KP), axis=1, keepdims=True)
    pt_ref[...] = jnp.sum(jnp.where(kio == am, cen, 0.0), axis=1,
                          keepdims=True)


def thresh_body(pf_ref, pt_ref, cls_ref, thre_ref, sc_ref):
    pf = pf_ref[...]
    cls = cls_ref[...]
    r = lax.broadcasted_iota(jnp.int32, pf.shape, 0)
    l = lax.broadcasted_iota(jnp.int32, pf.shape, 1)
    valid = (r * 128 + l) < N_SAMPLE
    thre_n = jnp.zeros_like(pf)
    flo = jnp.zeros_like(pf)
    for c in range(NUM_CLASSES):
        m = (cls == c) & valid
        cmin = jnp.min(jnp.where(m, pf, jnp.inf))
        thre_n = jnp.where(m, thre_ref[0, c], thre_n)
        flo = jnp.where(m, cmin - 10.0, flo)
    p = jnp.where(pf > thre_n, flo, pf)
    sc_ref[...] = pt_ref[...] - p


def gather_body(xrow_hbm, clsmap_hbm, idx_hbm, x_out, cls_out,
                idx_v, rows_v, qbuf, cout_v, sem):
    wid = lax.axis_index("s") * NC + lax.axis_index("c")
    base = wid * BPW
    pltpu.sync_copy(idx_hbm.at[pl.ds(base, BPW)], idx_v)
    for s in range(8):
        pltpu.async_copy(
            xrow_hbm.at[idx_v.at[pl.ds(s * 128, 128)]], rows_v, sem).wait()
        pltpu.sync_copy(rows_v, x_out.at[pl.ds(base + s * 128, 128), :])
    for q in range(4):
        pltpu.sync_copy(clsmap_hbm.at[pl.ds(q * QTR, QTR)], qbuf)

        def body(g, carry, q=q):
            iv = idx_v[pl.ds(g * 16, 16)]
            loc = iv - q * QTR
            m = (loc >= 0) & (loc < QTR)
            vals = plsc.load_gather(qbuf, [jnp.clip(loc, 0, QTR - 1)])
            prev = cout_v[pl.ds(g * 16, 16)]
            cout_v[pl.ds(g * 16, 16)] = jnp.where(m, vals, prev)
            return carry

        lax.fori_loop(0, BPW // 16, body, 0)
    pltpu.sync_copy(cout_v, cls_out.at[pl.ds(base, BPW)])


def scatter_body(score_hbm, idx_hbm, out_hbm, idx_v, sc_v, zone_v):
    wid = lax.axis_index("s") * NC + lax.axis_index("c")
    zlo = wid * ZONE

    def zero(i, carry):
        zone_v[pl.ds(i * 16, 16)] = jnp.zeros((16,), jnp.float32)
        return carry

    lax.fori_loop(0, ZONE // 16, zero, 0)

    @pl.when(wid < 2)
    def _():
        pltpu.sync_copy(idx_hbm, idx_v)
        pltpu.sync_copy(score_hbm, sc_v)

        def body(g, carry):
            iv = idx_v[pl.ds(g * 16, 16)]
            loc = iv - zlo
            sid = g * 16 + lax.iota(jnp.int32, 16)
            m = (loc >= 0) & (loc < ZONE) & (sid < N_SAMPLE)
            vals = sc_v[pl.ds(g * 16, 16)]
            plsc.store_scatter(zone_v, [jnp.clip(loc, 0, ZONE - 1)], vals,
                               mask=m)
            return carry

        lax.fori_loop(0, NP // 16, body, 0)

    pltpu.sync_copy(zone_v, out_hbm.at[pl.ds(zlo, ZONE)])


def _tc_calls():
    bp = 8192
    argmax_map = pl.pallas_call(
        argmax_body,
        grid=(pl.cdiv(HW, bp),),
        in_specs=[pl.BlockSpec((NUM_CLASSES, bp), lambda i: (0, i))],
        out_specs=pl.BlockSpec((1, bp), lambda i: (0, i)),
        out_shape=jax.ShapeDtypeStruct((1, HW), jnp.int32),
    )
    bt = 2048
    transpose = pl.pallas_call(
        transpose_body,
        grid=(pl.cdiv(HW, bt),),
        in_specs=[pl.BlockSpec((D, bt), lambda i: (0, i))],
        out_specs=pl.BlockSpec((bt, D), lambda i: (i, 0)),
        out_shape=jax.ShapeDtypeStruct((HW, D), jnp.float32),
    )
    params = pl.pallas_call(
        params_body,
        out_shape=(jax.ShapeDtypeStruct((NUM_CLASSES * K, 2 * D), jnp.float32),
                   jax.ShapeDtypeStruct((NUM_CLASSES, K), jnp.float32)),
    )
    bn = 2048
    gmm = pl.pallas_call(
        gmm_body,
        grid=(NP // bn,),
        in_specs=[
            pl.BlockSpec((bn, D), lambda i: (i, 0)),
            pl.BlockSpec((bn, 1), lambda i: (i, 0)),
            pl.BlockSpec((NUM_CLASSES * KP, 2 * D), lambda i: (0, 0)),
            pl.BlockSpec((NUM_CLASSES * KP, 2 * D), lambda i: (0, 0)),
            pl.BlockSpec((8, NUM_CLASSES * KP), lambda i: (0, 0)),
        ],
        out_specs=[pl.BlockSpec((bn, 1), lambda i: (i, 0)),
                   pl.BlockSpec((bn, 1), lambda i: (i, 0))],
        out_shape=(jax.ShapeDtypeStruct((NP, 1), jnp.float32),
                   jax.ShapeDtypeStruct((NP, 1), jnp.float32)),
    )
    thresh = pl.pallas_call(
        thresh_body,
        out_shape=jax.ShapeDtypeStruct((NP // 128, 128), jnp.float32),
    )
    return argmax_map, transpose, params, gmm, thresh


def _sc_calls():
    mesh = plsc.VectorSubcoreMesh(core_axis_name="c", subcore_axis_name="s")
    sc_params = pltpu.CompilerParams(needs_layout_passes=False)
    gather = functools.partial(
        pl.kernel, mesh=mesh, compiler_params=sc_params,
        out_type=(jax.ShapeDtypeStruct((NP, D), jnp.float32),
                  jax.ShapeDtypeStruct((NP,), jnp.int32)),
        scratch_types=[
            pltpu.VMEM((BPW,), jnp.int32),
            pltpu.VMEM((128, D), jnp.float32),
            pltpu.VMEM((QTR,), jnp.int32),
            pltpu.VMEM((BPW,), jnp.int32),
            pltpu.SemaphoreType.DMA,
        ],
    )(gather_body)
    scatter = functools.partial(
        pl.kernel, mesh=mesh, compiler_params=sc_params,
        out_type=jax.ShapeDtypeStruct((FULL,), jnp.float32),
        scratch_types=[
            pltpu.VMEM((NP,), jnp.int32),
            pltpu.VMEM((NP,), jnp.float32),
            pltpu.VMEM((ZONE,), jnp.float32),
        ],
    )(scatter_body)
    return gather, scatter


def kernel(features_tensor, outputs, sample_index, src_means, src_log_vars,
           src_log_weights, trg_means, trg_log_vars, trg_log_weights,
           trg_centers_2_src, gmm_thre):
    argmax_map, transpose, params, gmm, thresh = _tc_calls()
    gather, scatter = _sc_calls()

    feat2 = features_tensor.reshape(D, HW)
    out2 = outputs.reshape(NUM_CLASSES, HW)
    idxp = jnp.concatenate(
        [sample_index,
         jnp.full((NP - N_SAMPLE,), HW - 1, jnp.int32)])

    clsmap = argmax_map(out2).reshape(HW)
    xrow = transpose(feat2)
    x, cls = gather(xrow, clsmap, idxp)

    Ms, bs = params(src_means, src_log_vars, src_log_weights)
    Mt, bt = params(trg_means, trg_log_vars, trg_log_weights)
    def padk(M):
        return jnp.pad(M.reshape(NUM_CLASSES, K, 2 * D),
                       ((0, 0), (0, KP - K), (0, 0))).reshape(
                           NUM_CLASSES * KP, 2 * D)
    Msp, Mtp = padk(Ms), padk(Mt)
    def padb(b, fill):
        return jnp.pad(b, ((0, 0), (0, KP - K)),
                       constant_values=fill).reshape(NUM_CLASSES * KP)
    aux = jnp.zeros((8, NUM_CLASSES * KP), jnp.float32)
    aux = aux.at[0].set(padb(bs, NEG))
    aux = aux.at[1].set(padb(bt, NEG))
    aux = aux.at[2].set(padb(trg_centers_2_src, 0.0))

    pf, pt = gmm(x, cls.reshape(NP, 1), Msp, Mtp, aux)
    score = thresh(pf.reshape(NP // 128, 128), pt.reshape(NP // 128, 128),
                   cls.reshape(NP // 128, 128), gmm_thre.reshape(1, NUM_CLASSES))
    full = scatter(score.reshape(NP), idxp)
    return full.reshape(FULL_H, FULL_W)


# speedup vs baseline: 1.0344x; 1.0344x over previous
import functools

import jax
import jax.numpy as jnp
from jax import lax
from jax.experimental import pallas as pl
from jax.experimental.pallas import tpu as pltpu
from jax.experimental.pallas import tpu_sc as plsc

NUM_CLASSES = 19
K = 10
KP = 16
D = 512
H, W = 270, 480
HW = H * W
FULL_H, FULL_W = 1080, 1920
FULL = FULL_H * FULL_W
N_SAMPLE = 135 * 240
NP = 32768
NEG = -1e30

NC, NS = 2, 16
NW = NC * NS
BPW = NP // NW
QTR = HW // 4
ZONE = FULL // NW


def argmax_body(o_ref, cls_ref):
    x = o_ref[...]
    m = jnp.max(x, axis=0, keepdims=True)
    cid = lax.broadcasted_iota(jnp.int32, x.shape, 0)
    first = jnp.min(jnp.where(x == m, cid, NUM_CLASSES), axis=0, keepdims=True)
    cls_ref[...] = first


def transpose_body(f_ref, o_ref):
    o_ref[...] = f_ref[...].T


def params_body(m_ref, lv_ref, lw_ref, M_ref, b_ref):
    mm = m_ref[...].reshape(NUM_CLASSES * K, D)
    lv = lv_ref[...].reshape(NUM_CLASSES * K, D)
    inv = jnp.exp(-lv)
    M_ref[...] = jnp.concatenate([-0.5 * inv, mm * inv], axis=-1)
    lw = lw_ref[...]
    mx = jnp.max(lw, axis=-1, keepdims=True)
    lse = mx + jnp.log(jnp.sum(jnp.exp(lw - mx), axis=-1, keepdims=True))
    q3 = jnp.sum(mm * mm * inv, axis=-1).reshape(NUM_CLASSES, K)
    lsv = jnp.sum(lv, axis=-1).reshape(NUM_CLASSES, K)
    b_ref[...] = (lw - lse) - 0.5 * (D * jnp.log(2.0 * jnp.pi) + lsv + q3)


def gmm_body(x_ref, cls_ref, Ms_ref, Mt_ref, aux_ref, pf_ref, pt_ref):
    xb = x_ref[...]
    xx = jnp.concatenate([xb * xb, xb], axis=1)
    dn = (((1,), (1,)), ((), ()))
    comp_s = lax.dot_general(xx, Ms_ref[...], dn,
                             preferred_element_type=jnp.float32)
    comp_t = lax.dot_general(xx, Mt_ref[...], dn,
                             preferred_element_type=jnp.float32)
    comp_s = comp_s + aux_ref[0:1, :]
    comp_t = comp_t + aux_ref[1:2, :]
    clsb = cls_ref[...]
    bn = xb.shape[0]
    sel_s = jnp.zeros((bn, KP), jnp.float32)
    sel_t = jnp.zeros((bn, KP), jnp.float32)
    cen = jnp.zeros((bn, KP), jnp.float32)
    for c in range(NUM_CLASSES):
        m = clsb == c
        sl = slice(c * KP, (c + 1) * KP)
        sel_s = jnp.where(m, comp_s[:, sl], sel_s)
        sel_t = jnp.where(m, comp_t[:, sl], sel_t)
        cen = jnp.where(m, aux_ref[2:3, sl], cen)
    mxs = jnp.max(sel_s, axis=1, keepdims=True)
    pf_ref[...] = mxs + jnp.log(
        jnp.sum(jnp.exp(sel_s - mxs), axis=1, keepdims=True))
    mxt = jnp.max(sel_t, axis=1, keepdims=True)
    kio = lax.broadcasted_iota(jnp.int32, (bn, KP), 1)
    am = jnp.min(jnp.where(sel_t == mxt, kio, KP), axis=1, keepdims=True)
    pt_ref[...] = jnp.sum(jnp.where(kio == am, cen, 0.0), axis=1,
                          keepdims=True)


def thresh_body(pf_ref, pt_ref, cls_ref, thre_ref, sc_ref):
    pf = pf_ref[...]
    cls = cls_ref[...]
    r = lax.broadcasted_iota(jnp.int32, pf.shape, 0)
    l = lax.broadcasted_iota(jnp.int32, pf.shape, 1)
    valid = (r * 128 + l) < N_SAMPLE
    thre_n = jnp.zeros_like(pf)
    flo = jnp.zeros_like(pf)
    for c in range(NUM_CLASSES):
        m = (cls == c) & valid
        cmin = jnp.min(jnp.where(m, pf, jnp.inf))
        thre_n = jnp.where(m, thre_ref[0, c], thre_n)
        flo = jnp.where(m, cmin - 10.0, flo)
    p = jnp.where(pf > thre_n, flo, pf)
    sc_ref[...] = pt_ref[...] - p


def _first_ge(idx_v, bound):
    lo = jnp.int32(0)
    half = NP // 2
    while half >= 1:
        probe = idx_v[pl.ds(lo + (half - 1), 16)][0]
        lo = jnp.where(probe < bound, lo + half, lo)
        half //= 2
    return lo


def gather_x_body(xrow_hbm, idx_hbm, x_out, idx_v, rows_v, sem0, sem1):
    wid = lax.axis_index("s") * NC + lax.axis_index("c")
    base = wid * BPW
    pltpu.sync_copy(idx_hbm.at[pl.ds(base, BPW)], idx_v)
    sems = (sem0, sem1)
    cps = [None, None]
    cps[0] = pltpu.async_copy(
        xrow_hbm.at[idx_v.at[pl.ds(0, 64)]], rows_v.at[0], sems[0])
    for s in range(16):
        b = s % 2
        cps[b].wait()
        if s < 15:
            nb = (s + 1) % 2
            cps[nb] = pltpu.async_copy(
                xrow_hbm.at[idx_v.at[pl.ds((s + 1) * 64, 64)]],
                rows_v.at[nb], sems[nb])
        pltpu.sync_copy(rows_v.at[b], x_out.at[pl.ds(base + s * 64, 64), :])


def gather_cls_body(clsmap_hbm, idx_hbm, cls_out, idx_v, qbuf, cout_v):
    wid = lax.axis_index("s") * NC + lax.axis_index("c")
    base = wid * BPW
    pltpu.sync_copy(idx_hbm.at[pl.ds(base, BPW)], idx_v)
    for q in range(4):
        pltpu.sync_copy(clsmap_hbm.at[pl.ds(q * QTR, QTR)], qbuf)

        def body(g, carry, q=q):
            iv = idx_v[pl.ds(g * 16, 16)]
            loc = iv - q * QTR
            m = (loc >= 0) & (loc < QTR)
            vals = plsc.load_gather(qbuf, [jnp.clip(loc, 0, QTR - 1)])
            prev = cout_v[pl.ds(g * 16, 16)]
            cout_v[pl.ds(g * 16, 16)] = jnp.where(m, vals, prev)
            return carry

        lax.fori_loop(0, BPW // 16, body, 0)
    pltpu.sync_copy(cout_v, cls_out.at[pl.ds(base, BPW)])


SZONE = 4096


def scatter_body(score_hbm, idx_hbm, out_hbm, idx_v, sc_v, zone_v):
    wid = lax.axis_index("s") * NC + lax.axis_index("c")
    zlo = wid * SZONE

    def zero(i, carry):
        for u in range(8):
            zone_v[pl.ds(i * 128 + u * 16, 16)] = jnp.zeros((16,), jnp.float32)
        return carry

    lax.fori_loop(0, SZONE // 128, zero, 0)

    pltpu.sync_copy(idx_hbm, idx_v.at[pl.ds(0, NP)])
    pltpu.sync_copy(score_hbm, sc_v)
    g_lo = _first_ge(idx_v, zlo) // 16
    g_hi = (_first_ge(idx_v, zlo + SZONE) + 15) // 16

    def body(g, carry):
        iv = idx_v[pl.ds(g * 16, 16)]
        loc = iv - zlo
        sid = g * 16 + lax.iota(jnp.int32, 16)
        m = (loc >= 0) & (loc < SZONE) & (sid < N_SAMPLE)
        vals = sc_v[pl.ds(g * 16, 16)]
        plsc.store_scatter(zone_v, [jnp.clip(loc, 0, SZONE - 1)], vals,
                           mask=m)
        return carry

    lax.fori_loop(g_lo, g_hi, body, 0)
    pltpu.sync_copy(zone_v, out_hbm.at[pl.ds(zlo, SZONE)])


def _tc_calls():
    bp = 8192
    argmax_map = pl.pallas_call(
        argmax_body,
        grid=(pl.cdiv(HW, bp),),
        in_specs=[pl.BlockSpec((NUM_CLASSES, bp), lambda i: (0, i))],
        out_specs=pl.BlockSpec((1, bp), lambda i: (0, i)),
        out_shape=jax.ShapeDtypeStruct((1, HW), jnp.int32),
    )
    bt = 2048
    transpose = pl.pallas_call(
        transpose_body,
        grid=(pl.cdiv(HW, bt),),
        in_specs=[pl.BlockSpec((D, bt), lambda i: (0, i))],
        out_specs=pl.BlockSpec((bt, D), lambda i: (i, 0)),
        out_shape=jax.ShapeDtypeStruct((HW, D), jnp.float32),
    )
    params = pl.pallas_call(
        params_body,
        out_shape=(jax.ShapeDtypeStruct((NUM_CLASSES * K, 2 * D), jnp.float32),
                   jax.ShapeDtypeStruct((NUM_CLASSES, K), jnp.float32)),
    )
    bn = 2048
    gmm = pl.pallas_call(
        gmm_body,
        grid=(NP // bn,),
        in_specs=[
            pl.BlockSpec((bn, D), lambda i: (i, 0)),
            pl.BlockSpec((bn, 1), lambda i: (i, 0)),
            pl.BlockSpec((NUM_CLASSES * KP, 2 * D), lambda i: (0, 0)),
            pl.BlockSpec((NUM_CLASSES * KP, 2 * D), lambda i: (0, 0)),
            pl.BlockSpec((8, NUM_CLASSES * KP), lambda i: (0, 0)),
        ],
        out_specs=[pl.BlockSpec((bn, 1), lambda i: (i, 0)),
                   pl.BlockSpec((bn, 1), lambda i: (i, 0))],
        out_shape=(jax.ShapeDtypeStruct((NP, 1), jnp.float32),
                   jax.ShapeDtypeStruct((NP, 1), jnp.float32)),
    )
    thresh = pl.pallas_call(
        thresh_body,
        out_shape=jax.ShapeDtypeStruct((NP // 128, 128), jnp.float32),
    )
    return argmax_map, transpose, params, gmm, thresh


def _sc_calls():
    mesh = plsc.VectorSubcoreMesh(core_axis_name="c", subcore_axis_name="s")
    sc_params = pltpu.CompilerParams(needs_layout_passes=False)
    gather_x = functools.partial(
        pl.kernel, mesh=mesh, compiler_params=sc_params,
        out_type=jax.ShapeDtypeStruct((NP, D), jnp.float32),
        scratch_types=[
            pltpu.VMEM((BPW,), jnp.int32),
            pltpu.VMEM((2, 64, D), jnp.float32),
            pltpu.SemaphoreType.DMA,
            pltpu.SemaphoreType.DMA,
        ],
    )(gather_x_body)
    gather_cls = functools.partial(
        pl.kernel, mesh=mesh, compiler_params=sc_params,
        out_type=jax.ShapeDtypeStruct((NP,), jnp.int32),
        scratch_types=[
            pltpu.VMEM((BPW,), jnp.int32),
            pltpu.VMEM((QTR,), jnp.int32),
            pltpu.VMEM((BPW,), jnp.int32),
        ],
    )(gather_cls_body)
    scatter = functools.partial(
        pl.kernel, mesh=mesh, compiler_params=sc_params,
        out_type=jax.ShapeDtypeStruct((NW * SZONE,), jnp.float32),
        scratch_types=[
            pltpu.VMEM((NP + 16,), jnp.int32),
            pltpu.VMEM((NP,), jnp.float32),
            pltpu.VMEM((SZONE,), jnp.float32),
        ],
    )(scatter_body)
    return gather_x, gather_cls, scatter


def kernel(features_tensor, outputs, sample_index, src_means, src_log_vars,
           src_log_weights, trg_means, trg_log_vars, trg_log_weights,
           trg_centers_2_src, gmm_thre):
    argmax_map, transpose, params, gmm, thresh = _tc_calls()
    gather_x, gather_cls, scatter = _sc_calls()

    feat2 = features_tensor.reshape(D, HW)
    out2 = outputs.reshape(NUM_CLASSES, HW)
    idxp = jnp.concatenate(
        [sample_index,
         jnp.full((NP - N_SAMPLE,), HW - 1, jnp.int32)])

    clsmap = argmax_map(out2).reshape(HW)
    cls = gather_cls(clsmap, idxp)
    xrow = transpose(feat2)
    x = gather_x(xrow, idxp)

    Ms, bs = params(src_means, src_log_vars, src_log_weights)
    Mt, bt = params(trg_means, trg_log_vars, trg_log_weights)
    def padk(M):
        return jnp.pad(M.reshape(NUM_CLASSES, K, 2 * D),
                       ((0, 0), (0, KP - K), (0, 0))).reshape(
                           NUM_CLASSES * KP, 2 * D)
    Msp, Mtp = padk(Ms), padk(Mt)
    def padb(b, fill):
        return jnp.pad(b, ((0, 0), (0, KP - K)),
                       constant_values=fill).reshape(NUM_CLASSES * KP)
    aux = jnp.zeros((8, NUM_CLASSES * KP), jnp.float32)
    aux = aux.at[0].set(padb(bs, NEG))
    aux = aux.at[1].set(padb(bt, NEG))
    aux = aux.at[2].set(padb(trg_centers_2_src, 0.0))

    pf, pt = gmm(x, cls.reshape(NP, 1), Msp, Mtp, aux)
    score = thresh(pf.reshape(NP // 128, 128), pt.reshape(NP // 128, 128),
                   cls.reshape(NP // 128, 128), gmm_thre.reshape(1, NUM_CLASSES))
    prefix = scatter(score.reshape(NP), idxp)
    full = jnp.concatenate([prefix[:HW], jnp.zeros(FULL - HW, jnp.float32)])
    return full.reshape(FULL_H, FULL_W)
